# baseline (speedup 1.0000x reference)
import jax
import jax.numpy as jnp
from jax import lax
from jax.experimental import pallas as pl
from jax.experimental.pallas import tpu as pltpu
from jax.experimental.pallas import tpu_sc as plsc

DIM = 32
WPD = DIM // 2
L = 16
NC = 2
NS = 16
NW = NC * NS
BATCH = 16384
BPW = BATCH // NW
CHUNK = 128
NCHUNK = BPW // CHUNK


def _mf_body(uid_hbm, iid_hbm, ut_hbm, it_hbm, out_hbm,
             uidx_v, iidx_v, urows_v, irows_v, out_v, sem):
    wid = lax.axis_index("s") * NC + lax.axis_index("c")
    row0 = wid * NCHUNK

    pltpu.sync_copy(uid_hbm.at[pl.ds(row0, NCHUNK)], uidx_v)
    pltpu.sync_copy(iid_hbm.at[pl.ds(row0, NCHUNK)], iidx_v)

    for j in range(NCHUNK):
        for k in range(CHUNK // L):
            sl = pl.ds(k * L, L)
            uidx_v[j, sl] = uidx_v[j, sl] - 1
            iidx_v[j, sl] = iidx_v[j, sl] - 1

    copies = []
    for j in range(NCHUNK):
        copies.append(pltpu.async_copy(
            ut_hbm.at[uidx_v.at[j]], urows_v.at[pl.ds(j * CHUNK, CHUNK)], sem))
        copies.append(pltpu.async_copy(
            it_hbm.at[iidx_v.at[j]], irows_v.at[pl.ds(j * CHUNK, CHUNK)], sem))
    for c in copies:
        c.wait()

    lane = lax.iota(jnp.int32, L)

    def group(g, carry):
        acc = jnp.zeros((L,), jnp.float32)
        for e16 in range(L):
            e = g * L + e16
            p = urows_v[e, :] * irows_v[e, :]
            a, b = plsc.unpack(p, format=plsc.PackFormat.INTERLEAVED)
            tot = jnp.sum(a + b)
            acc = jnp.where(lane == e16, tot, acc)
        out_v[pl.ds(g * L, L)] = acc
        return carry

    lax.fori_loop(0, BPW // L, group, 0)

    pltpu.sync_copy(out_v, out_hbm.at[pl.ds(wid * BPW, BPW)])


def kernel(user_id, item_id, user_table, item_table):
    uid2 = user_id.reshape(NW * NCHUNK, CHUNK)
    iid2 = item_id.reshape(NW * NCHUNK, CHUNK)
    mesh = plsc.VectorSubcoreMesh(core_axis_name="c", subcore_axis_name="s")
    f = pl.kernel(
        _mf_body,
        mesh=mesh,
        compiler_params=pltpu.CompilerParams(
            needs_layout_passes=False, use_tc_tiling_on_sc=False),
        out_type=jax.ShapeDtypeStruct((BATCH,), jnp.float32),
        scratch_types=[
            pltpu.VMEM((NCHUNK, CHUNK), jnp.int32),
            pltpu.VMEM((NCHUNK, CHUNK), jnp.int32),
            pltpu.VMEM((BPW, DIM), jnp.bfloat16),
            pltpu.VMEM((BPW, DIM), jnp.bfloat16),
            pltpu.VMEM((BPW,), jnp.float32),
            pltpu.SemaphoreType.DMA,
        ],
    )
    return f(uid2, iid2,
             user_table.astype(jnp.bfloat16), item_table.astype(jnp.bfloat16))


# speedup vs baseline: 1.1562x; 1.1562x over previous
import jax
import jax.numpy as jnp
from jax import lax
from jax.experimental import pallas as pl
from jax.experimental.pallas import tpu as pltpu
from jax.experimental.pallas import tpu_sc as plsc

DIM = 32
L = 16
NC = 2
NS = 16
NW = NC * NS
BATCH = 16384
BPW = BATCH // NW
CHUNK = 128
NCHUNK = BPW // CHUNK


def _mf_body(uid_hbm, iid_hbm, ut_hbm, it_hbm, out_hbm,
             uidx_v, iidx_v, urows_v, irows_v, out_v, sem):
    wid = lax.axis_index("s") * NC + lax.axis_index("c")
    row0 = wid * NCHUNK

    pltpu.sync_copy(uid_hbm.at[pl.ds(row0, NCHUNK)], uidx_v)
    pltpu.sync_copy(iid_hbm.at[pl.ds(row0, NCHUNK)], iidx_v)

    for j in range(NCHUNK):
        for k in range(CHUNK // L):
            sl = pl.ds(k * L, L)
            uidx_v[j, sl] = uidx_v[j, sl] - 1
            iidx_v[j, sl] = iidx_v[j, sl] - 1

    copies = []
    for j in range(NCHUNK):
        copies.append(pltpu.async_copy(
            ut_hbm.at[uidx_v.at[j]], urows_v.at[pl.ds(j * CHUNK, CHUNK)], sem))
        copies.append(pltpu.async_copy(
            it_hbm.at[iidx_v.at[j]], irows_v.at[pl.ds(j * CHUNK, CHUNK)], sem))
    for c in copies:
        c.wait()

    lane = lax.iota(jnp.int32, L)

    def group(g, carry):
        row = g * L + lane
        acc = jnp.zeros((L,), jnp.float32)
        for d in range(DIM):
            col = jnp.full((L,), d, jnp.int32)
            cu = plsc.load_gather(urows_v, [row, col])
            ci = plsc.load_gather(irows_v, [row, col])
            acc = acc + cu * ci
        out_v[pl.ds(g * L, L)] = acc
        return carry

    lax.fori_loop(0, BPW // L, group, 0)

    pltpu.sync_copy(out_v, out_hbm.at[pl.ds(wid * BPW, BPW)])


def kernel(user_id, item_id, user_table, item_table):
    uid2 = user_id.reshape(NW * NCHUNK, CHUNK)
    iid2 = item_id.reshape(NW * NCHUNK, CHUNK)
    mesh = plsc.VectorSubcoreMesh(core_axis_name="c", subcore_axis_name="s")
    f = pl.kernel(
        _mf_body,
        mesh=mesh,
        compiler_params=pltpu.CompilerParams(
            needs_layout_passes=False, use_tc_tiling_on_sc=False),
        out_type=jax.ShapeDtypeStruct((BATCH,), jnp.float32),
        scratch_types=[
            pltpu.VMEM((NCHUNK, CHUNK), jnp.int32),
            pltpu.VMEM((NCHUNK, CHUNK), jnp.int32),
            pltpu.VMEM((BPW, DIM), jnp.float32),
            pltpu.VMEM((BPW, DIM), jnp.float32),
            pltpu.VMEM((BPW,), jnp.float32),
            pltpu.SemaphoreType.DMA,
        ],
    )
    return f(uid2, iid2, user_table, item_table)


# speedup vs baseline: 4.5271x; 3.9153x over previous
import jax
import jax.numpy as jnp
from jax import lax
from jax.experimental import pallas as pl
from jax.experimental.pallas import tpu as pltpu
from jax.experimental.pallas import tpu_sc as plsc

DIM = 32
L = 16
NC = 2
NS = 16
NW = NC * NS
BATCH = 16384
BPW = BATCH // NW
STRIDE = 1024
NBUF = 12
BLK = 128
NGRP = BPW // L


def _mf_body(uid_hbm, iid_hbm, ut_hbm, it_hbm, out_hbm,
             uids_v, iids_v, ubuf_v, ibuf_v, out_v, sem):
    wid = lax.axis_index("s") * NC + lax.axis_index("c")

    pltpu.sync_copy(uid_hbm.at[pl.ds(wid * STRIDE, STRIDE)], uids_v)
    pltpu.sync_copy(iid_hbm.at[pl.ds(wid * STRIDE, STRIDE)], iids_v)

    def fire(slot, ru, ri):
        cu = pl.multiple_of((ru >> 7) * BLK, BLK)
        ci = pl.multiple_of((ri >> 7) * BLK, BLK)
        pltpu.async_copy(ut_hbm.at[:, pl.ds(cu, BLK)], ubuf_v.at[slot], sem)
        pltpu.async_copy(it_hbm.at[:, pl.ds(ci, BLK)], ibuf_v.at[slot], sem)

    def drain_one():
        pltpu.make_async_copy(ut_hbm.at[:, pl.ds(0, BLK)], ubuf_v.at[0], sem).wait()
        pltpu.make_async_copy(it_hbm.at[:, pl.ds(0, BLK)], ibuf_v.at[0], sem).wait()

    lane = lax.iota(jnp.int32, L)

    uv0 = uids_v[pl.ds(0, L)]
    iv0 = iids_v[pl.ds(0, L)]
    for p in range(NBUF):
        fire(p, uv0[p] - 1, iv0[p] - 1)

    def make_group(do_fire):
        def body(g, carry):
            base = g * L
            uv_g = uids_v[pl.ds(base, L)]
            iv_g = iids_v[pl.ds(base, L)]
            if do_fire:
                uv_n = uids_v[pl.ds(base + L, L)]
                iv_n = iids_v[pl.ds(base + L, L)]
            acc = jnp.zeros((L,), jnp.float32)
            for k in range(L):
                drain_one()
                e = base + k
                slot = jnp.full((L,), lax.rem(e, NBUF), jnp.int32)
                ru = uv_g[k] - 1
                ri = iv_g[k] - 1
                colu = jnp.full((L,), ru & 127, jnp.int32)
                coli = jnp.full((L,), ri & 127, jnp.int32)
                hi = lane + L
                u0 = plsc.load_gather(ubuf_v, [slot, lane, colu])
                u1 = plsc.load_gather(ubuf_v, [slot, hi, colu])
                i0 = plsc.load_gather(ibuf_v, [slot, lane, coli])
                i1 = plsc.load_gather(ibuf_v, [slot, hi, coli])
                acc = jnp.where(lane == k, jnp.sum(u0 * i0 + u1 * i1), acc)
                if do_fire:
                    ru2 = (uv_g[k + NBUF] if k + NBUF < L else uv_n[k + NBUF - L]) - 1
                    ri2 = (iv_g[k + NBUF] if k + NBUF < L else iv_n[k + NBUF - L]) - 1
                    fire(lax.rem(e + NBUF, NBUF), ru2, ri2)
            out_v[pl.ds(base, L)] = acc
            return carry
        return body

    lax.fori_loop(0, NGRP, make_group(True), 0)

    for _ in range(NBUF):
        drain_one()

    pltpu.sync_copy(out_v, out_hbm.at[pl.ds(wid * STRIDE, STRIDE)])


def kernel(user_id, item_id, user_table, item_table):
    pad = jnp.ones((NW, STRIDE - BPW), jnp.int32)
    uid_pad = jnp.concatenate([user_id.reshape(NW, BPW), pad], axis=1).reshape(-1)
    iid_pad = jnp.concatenate([item_id.reshape(NW, BPW), pad], axis=1).reshape(-1)
    mesh = plsc.VectorSubcoreMesh(core_axis_name="c", subcore_axis_name="s")
    f = pl.kernel(
        _mf_body,
        mesh=mesh,
        compiler_params=pltpu.CompilerParams(needs_layout_passes=False),
        out_type=jax.ShapeDtypeStruct((NW * STRIDE,), jnp.float32),
        scratch_types=[
            pltpu.VMEM((STRIDE,), jnp.int32),
            pltpu.VMEM((STRIDE,), jnp.int32),
            pltpu.VMEM((NBUF, DIM, BLK), jnp.float32),
            pltpu.VMEM((NBUF, DIM, BLK), jnp.float32),
            pltpu.VMEM((STRIDE,), jnp.float32),
            pltpu.SemaphoreType.DMA,
        ],
    )
    out = f(uid_pad, iid_pad, user_table.T, item_table.T)
    return out.reshape(NW, STRIDE)[:, :BPW].reshape(BATCH)
